# Initial kernel scaffold; baseline (speedup 1.0000x reference)
import functools

import jax
import jax.numpy as jnp
from jax import lax
from jax.experimental import pallas as pl
from jax.experimental.pallas import tpu as pltpu
from jax.experimental.pallas import tpu_sc as plsc

NC = 2
NS = 16
NW = NC * NS
L = 16

C = 80


def _mm_body(x_ref, wl_ref, wr_ref, xl_ref, xr_ref):
    xb = x_ref[...]
    xl_ref[...] = jnp.dot(xb, wl_ref[...], preferred_element_type=jnp.float32)
    xr_ref[...] = jnp.dot(xb, wr_ref[...], preferred_element_type=jnp.float32)


def _dual_matmul(x, W_l, W_r):
    n, d = x.shape
    bn = 400
    grid = (n // bn,)
    return pl.pallas_call(
        _mm_body,
        grid=grid,
        in_specs=[
            pl.BlockSpec((bn, d), lambda i: (i, 0)),
            pl.BlockSpec((d, d), lambda i: (0, 0)),
            pl.BlockSpec((d, d), lambda i: (0, 0)),
        ],
        out_specs=[
            pl.BlockSpec((bn, d), lambda i: (i, 0)),
            pl.BlockSpec((bn, d), lambda i: (i, 0)),
        ],
        out_shape=[
            jax.ShapeDtypeStruct((n, d), jnp.float32),
            jax.ShapeDtypeStruct((n, d), jnp.float32),
        ],
    )(x, W_l, W_r)


def _sc_edge(xl, xr, att, src, dst):
    n, d = xl.shape
    e = src.shape[0]
    dk = d // L
    ew = e // NW
    nchunk = ew // C
    nzc = n // C
    mesh = plsc.VectorSubcoreMesh(core_axis_name="c", subcore_axis_name="s")

    @functools.partial(
        pl.kernel,
        out_type=[
            jax.ShapeDtypeStruct((NC, n, d), jnp.float32),
            jax.ShapeDtypeStruct((NC, n, L), jnp.float32),
        ],
        mesh=mesh,
        scratch_types=[
            pltpu.VMEM((C,), jnp.int32),
            pltpu.VMEM((C,), jnp.int32),
            pltpu.VMEM((C, d), jnp.float32),
            pltpu.VMEM((C, d), jnp.float32),
            pltpu.VMEM((C, L), jnp.float32),
            pltpu.VMEM((d,), jnp.float32),
            pltpu.VMEM_SHARED((n, d), jnp.float32),
            pltpu.VMEM_SHARED((n, L), jnp.float32),
            pltpu.SemaphoreType.DMA,
            pltpu.SemaphoreType.DMA,
        ],
    )
    def k(xl_hbm, xr_hbm, att_hbm, src_hbm, dst_hbm, accr_hbm, accs_hbm,
          srcv, dstv, xlv, xrv, exv, attv, acc_sh, accs_sh, sem1, sem2):
        cid = lax.axis_index("c")
        sid = lax.axis_index("s")
        wid = sid * NC + cid

        def zbuf(i, _):
            for kk in range(dk):
                xlv[i, pl.ds(kk * L, L)] = jnp.zeros((L,), jnp.float32)
            exv[i, :] = jnp.zeros((L,), jnp.float32)
            return 0

        lax.fori_loop(0, C, zbuf, 0)

        def zacc(j, _):
            chunk = sid + j * NS

            @pl.when(chunk < nzc)
            def _():
                pltpu.sync_copy(xlv, acc_sh.at[pl.ds(chunk * C, C)])
                pltpu.sync_copy(exv, accs_sh.at[pl.ds(chunk * C, C)])

            return 0

        lax.fori_loop(0, (nzc + NS - 1) // NS, zacc, 0)
        plsc.subcore_barrier()

        pltpu.sync_copy(att_hbm, attv)
        att_regs = [attv[pl.ds(kk * L, L)] for kk in range(dk)]
        ones = jnp.full((L,), 1.0, jnp.float32)
        zeros = jnp.zeros((L,), jnp.float32)
        lane = lax.iota(jnp.int32, L)

        def chunk_body(ci, _):
            base = wid * ew + ci * C
            pltpu.sync_copy(src_hbm.at[pl.ds(base, C)], srcv)
            pltpu.sync_copy(dst_hbm.at[pl.ds(base, C)], dstv)
            cp1 = pltpu.async_copy(xl_hbm.at[srcv], xlv, sem1)
            cp2 = pltpu.async_copy(xr_hbm.at[dstv], xrv, sem2)
            cp1.wait()
            cp2.wait()

            def edge_body(ei, _):
                parts = None
                xls = []
                for kk in range(dk):
                    a = xlv[ei, pl.ds(kk * L, L)]
                    b = xrv[ei, pl.ds(kk * L, L)]
                    v = a + b
                    lr = 0.6 * v + 0.4 * jnp.abs(v)
                    t = att_regs[kk] * lr
                    parts = t if parts is None else parts + t
                    xls.append(a)
                tot = jnp.sum(parts)
                ex = jnp.exp(jnp.full((L,), tot))
                exv[ei, :] = jnp.where(lane == 0, ex,
                                       jnp.where(lane == 1, ones, zeros))
                for kk in range(dk):
                    xlv[ei, pl.ds(kk * L, L)] = xls[kk] * ex
                return 0

            lax.fori_loop(0, C, edge_body, 0)
            pltpu.sync_copy(xlv, acc_sh.at[dstv], add=True)
            pltpu.sync_copy(exv, accs_sh.at[dstv], add=True)
            return 0

        lax.fori_loop(0, nchunk, chunk_body, 0)
        plsc.subcore_barrier()

        def dump(j, _):
            chunk = sid + j * NS

            @pl.when(chunk < nzc)
            def _():
                pltpu.sync_copy(acc_sh.at[pl.ds(chunk * C, C)],
                                accr_hbm.at[cid, pl.ds(chunk * C, C)])
                pltpu.sync_copy(accs_sh.at[pl.ds(chunk * C, C)],
                                accs_hbm.at[cid, pl.ds(chunk * C, C)])

            return 0

        lax.fori_loop(0, (nzc + NS - 1) // NS, dump, 0)

    return k(xl, xr, att, src, dst)


def _fin_body(accr_ref, accs_ref, bias_ref, out_ref):
    r = accr_ref[0] + accr_ref[1]
    s = accs_ref[0] + accs_ref[1]
    denom = s[:, 0:1] + 1e-16
    count = jnp.maximum(s[:, 1:2], 1.0)
    out_ref[...] = r / (denom * count) + bias_ref[...]


def _finalize(accr, accs, bias2d):
    _, n, d = accr.shape
    bn = 400
    return pl.pallas_call(
        _fin_body,
        grid=(n // bn,),
        in_specs=[
            pl.BlockSpec((2, bn, d), lambda i: (0, i, 0)),
            pl.BlockSpec((2, bn, L), lambda i: (0, i, 0)),
            pl.BlockSpec((1, d), lambda i: (0, 0)),
        ],
        out_specs=pl.BlockSpec((bn, d), lambda i: (i, 0)),
        out_shape=jax.ShapeDtypeStruct((n, d), jnp.float32),
    )(accr, accs, bias2d)


def kernel(x, edge_index, W_l, W_r, att, bias):
    src = edge_index[0].astype(jnp.int32)
    dst = edge_index[1].astype(jnp.int32)
    xl, xr = _dual_matmul(x, W_l, W_r)
    accr, accs = _sc_edge(xl, xr, att, src, dst)
    return _finalize(accr, accs, bias.reshape(1, -1))


# speedup vs baseline: 4.9915x; 4.9915x over previous
import functools

import jax
import jax.numpy as jnp
from jax import lax
from jax.experimental import pallas as pl
from jax.experimental.pallas import tpu as pltpu
from jax.experimental.pallas import tpu_sc as plsc

NC = 2
NS = 16
NW = NC * NS
L = 16

C = 80

_TAKE_DNUMS = lax.GatherDimensionNumbers(
    offset_dims=(), collapsed_slice_dims=(0,), start_index_map=(0,))


def _lane_take(v, perm):
    return lax.gather(v, perm[:, None], _TAKE_DNUMS, slice_sizes=(1,),
                      mode=lax.GatherScatterMode.PROMISE_IN_BOUNDS)


def _mm_body(x_ref, wl_ref, wr_ref, xl_ref, xr_ref):
    xb = x_ref[...]
    xl_ref[...] = jnp.dot(xb, wl_ref[...], preferred_element_type=jnp.float32)
    xr_ref[...] = jnp.dot(xb, wr_ref[...], preferred_element_type=jnp.float32)


def _dual_matmul(x, W_l, W_r):
    n, d = x.shape
    bn = 400
    grid = (n // bn,)
    return pl.pallas_call(
        _mm_body,
        grid=grid,
        in_specs=[
            pl.BlockSpec((bn, d), lambda i: (i, 0)),
            pl.BlockSpec((d, d), lambda i: (0, 0)),
            pl.BlockSpec((d, d), lambda i: (0, 0)),
        ],
        out_specs=[
            pl.BlockSpec((bn, d), lambda i: (i, 0)),
            pl.BlockSpec((bn, d), lambda i: (i, 0)),
        ],
        out_shape=[
            jax.ShapeDtypeStruct((n, d), jnp.float32),
            jax.ShapeDtypeStruct((n, d), jnp.float32),
        ],
    )(x, W_l, W_r)


def _sc_edge(xl, xr, att, src, dst):
    n, d = xl.shape
    e = src.shape[0]
    dk = d // L
    dw = d + L
    half = n // NC
    W = 5376
    TRASH = half + 64
    rpt = W // NS
    ZD = 48
    ew = e // NS
    nchunk = ew // C
    ng = C // L
    mesh = plsc.VectorSubcoreMesh(core_axis_name="c", subcore_axis_name="s")

    @functools.partial(
        pl.kernel,
        out_type=[
            jax.ShapeDtypeStruct((NC, W, d), jnp.float32),
            jax.ShapeDtypeStruct((NC, NS, W), jnp.float32),
            jax.ShapeDtypeStruct((NC, NS, W), jnp.float32),
        ],
        mesh=mesh,
        scratch_types=[
            pltpu.VMEM((C,), jnp.int32),
            pltpu.VMEM((C,), jnp.int32),
            pltpu.VMEM((C, d), jnp.float32),
            pltpu.VMEM((C, d), jnp.float32),
            pltpu.VMEM((C, L), jnp.float32),
            pltpu.VMEM((d,), jnp.float32),
            pltpu.VMEM((W,), jnp.float32),
            pltpu.VMEM((W,), jnp.float32),
            pltpu.VMEM_SHARED((W, d), jnp.float32),
            pltpu.SemaphoreType.DMA,
            pltpu.SemaphoreType.DMA,
        ],
    )
    def k(xl_hbm, xr_hbm, att_hbm, src_hbm, dst_hbm, accr_hbm, den_hbm,
          cnt_hbm, srcv, dstv, xlv, xrv, exv, attv, denomv, countv, acc_sh,
          sem1, sem2):
        cid = lax.axis_index("c")
        sid = lax.axis_index("s")
        nbase = cid * half

        @pl.loop(0, ZD)
        def zbuf(i):
            for kk in range(dk):
                xlv[i, pl.ds(kk * L, L)] = jnp.zeros((L,), jnp.float32)

        @pl.loop(0, rpt // ZD)
        def zacc(j):
            off = sid * rpt + j * ZD
            pltpu.sync_copy(xlv.at[pl.ds(0, ZD)], acc_sh.at[pl.ds(off, ZD)])

        @pl.loop(0, W // L)
        def zdc(i):
            denomv[pl.ds(i * L, L)] = jnp.zeros((L,), jnp.float32)
            countv[pl.ds(i * L, L)] = jnp.zeros((L,), jnp.float32)

        plsc.subcore_barrier()

        pltpu.sync_copy(att_hbm, attv)
        att_regs = [attv[pl.ds(kk * L, L)] for kk in range(dk)]
        ones = jnp.full((L,), 1.0, jnp.float32)
        zeros = jnp.zeros((L,), jnp.float32)
        lane = lax.iota(jnp.int32, L)
        trash = jnp.full((L,), TRASH, jnp.int32)

        @pl.loop(0, nchunk)
        def chunk_body(ci):
            base = sid * ew + ci * C
            pltpu.sync_copy(src_hbm.at[pl.ds(base, C)], srcv)
            pltpu.sync_copy(dst_hbm.at[pl.ds(base, C)], dstv)
            cp1 = pltpu.async_copy(xl_hbm.at[srcv], xlv, sem1)
            cp2 = pltpu.async_copy(xr_hbm.at[dstv], xrv, sem2)
            cp1.wait()
            cp2.wait()

            for g in range(ng):
                d16 = dstv[pl.ds(g * L, L)]
                loc = d16 - nbase
                own = (d16 >= nbase) & (loc < half)
                dstv[pl.ds(g * L, L)] = jnp.where(own, loc, trash)

            @pl.loop(0, C)
            def edge_body(ei):
                parts = None
                xls = []
                for kk in range(dk):
                    a = xlv[ei, pl.ds(kk * L, L)]
                    b = xrv[ei, pl.ds(kk * L, L)]
                    v = a + b
                    lr = 0.6 * v + 0.4 * jnp.abs(v)
                    t = att_regs[kk] * lr
                    parts = t if parts is None else parts + t
                    xls.append(a)
                s = parts
                for sh in (8, 4, 2, 1):
                    perm = jnp.bitwise_xor(lane, sh)
                    s = s + _lane_take(s, perm)
                ex = jnp.exp(s)
                exv[ei, :] = ex
                for kk in range(dk):
                    xlv[ei, pl.ds(kk * L, L)] = xls[kk] * ex

            for g in range(ng):
                d16 = dstv[pl.ds(g * L, L)]
                for j in range(L):
                    exj = exv[g * L + j, :]
                    loc = d16[j]
                    bs = lax.div(loc, L) * L
                    msk = lane == (loc - bs)
                    denomv[pl.ds(bs, L)] = (denomv[pl.ds(bs, L)]
                                            + jnp.where(msk, exj, zeros))
                    countv[pl.ds(bs, L)] = (countv[pl.ds(bs, L)]
                                            + jnp.where(msk, ones, zeros))

            pltpu.sync_copy(xlv, acc_sh.at[dstv], add=True)

        plsc.subcore_barrier()

        @pl.loop(0, rpt // ZD)
        def dump(j):
            off = sid * rpt + j * ZD
            pltpu.sync_copy(acc_sh.at[pl.ds(off, ZD)],
                            accr_hbm.at[cid, pl.ds(off, ZD)])

        pltpu.sync_copy(denomv, den_hbm.at[cid, sid])
        pltpu.sync_copy(countv, cnt_hbm.at[cid, sid])

    return k(xl, xr, att, src, dst)


def _fin_body(accr_ref, den_ref, cnt_ref, bias_ref, out_ref):
    r = accr_ref[0]
    denom = jnp.sum(den_ref[0], axis=1)[:, None] + 1e-16
    count = jnp.maximum(jnp.sum(cnt_ref[0], axis=1), 1.0)[:, None]
    out_ref[...] = r / (denom * count) + bias_ref[...]


def _finalize(accr, den_t, cnt_t, bias2d):
    n = 10000
    d = accr.shape[-1]
    bn = 200
    nb = (n // NC) // bn
    return pl.pallas_call(
        _fin_body,
        grid=(n // bn,),
        in_specs=[
            pl.BlockSpec((1, bn, d),
                         lambda i: (lax.div(i, nb), lax.rem(i, nb), 0)),
            pl.BlockSpec((1, bn, NS),
                         lambda i: (lax.div(i, nb), lax.rem(i, nb), 0)),
            pl.BlockSpec((1, bn, NS),
                         lambda i: (lax.div(i, nb), lax.rem(i, nb), 0)),
            pl.BlockSpec((1, d), lambda i: (0, 0)),
        ],
        out_specs=pl.BlockSpec((bn, d), lambda i: (i, 0)),
        out_shape=jax.ShapeDtypeStruct((n, d), jnp.float32),
    )(accr, den_t, cnt_t, bias2d)


def kernel(x, edge_index, W_l, W_r, att, bias):
    src = edge_index[0].astype(jnp.int32)
    dst = edge_index[1].astype(jnp.int32)
    xl, xr = _dual_matmul(x, W_l, W_r)
    accr, den, cnt = _sc_edge(xl, xr, att, src, dst)
    den_t = jnp.transpose(den, (0, 2, 1))
    cnt_t = jnp.transpose(cnt, (0, 2, 1))
    return _finalize(accr, den_t, cnt_t, bias.reshape(1, -1))


# speedup vs baseline: 6.7830x; 1.3589x over previous
import functools

import jax
import jax.numpy as jnp
from jax import lax
from jax.experimental import pallas as pl
from jax.experimental.pallas import tpu as pltpu
from jax.experimental.pallas import tpu_sc as plsc

NC = 2
NS = 16
NW = NC * NS
L = 16

C = 80

_TAKE_DNUMS = lax.GatherDimensionNumbers(
    offset_dims=(), collapsed_slice_dims=(0,), start_index_map=(0,))


def _lane_take(v, perm):
    return lax.gather(v, perm[:, None], _TAKE_DNUMS, slice_sizes=(1,),
                      mode=lax.GatherScatterMode.PROMISE_IN_BOUNDS)


def _mm_body(x_ref, wl_ref, wr_ref, xl_ref, xr_ref):
    xb = x_ref[...]
    xl_ref[...] = jnp.dot(xb, wl_ref[...], preferred_element_type=jnp.float32)
    xr_ref[...] = jnp.dot(xb, wr_ref[...], preferred_element_type=jnp.float32)


def _dual_matmul(x, W_l, W_r):
    n, d = x.shape
    bn = 400
    grid = (n // bn,)
    return pl.pallas_call(
        _mm_body,
        grid=grid,
        in_specs=[
            pl.BlockSpec((bn, d), lambda i: (i, 0)),
            pl.BlockSpec((d, d), lambda i: (0, 0)),
            pl.BlockSpec((d, d), lambda i: (0, 0)),
        ],
        out_specs=[
            pl.BlockSpec((bn, d), lambda i: (i, 0)),
            pl.BlockSpec((bn, d), lambda i: (i, 0)),
        ],
        out_shape=[
            jax.ShapeDtypeStruct((n, d), jnp.float32),
            jax.ShapeDtypeStruct((n, d), jnp.float32),
        ],
    )(x, W_l, W_r)


def _sc_edge(xl, xr, att, src, dst):
    n, d = xl.shape
    e = src.shape[0]
    dk = d // L
    dw = d + L
    half = n // NC
    W = 5376
    TRASH = half + 64
    rpt = W // NS
    ZD = 48
    ew = e // NS
    nchunk = ew // C
    ng = C // L
    mesh = plsc.VectorSubcoreMesh(core_axis_name="c", subcore_axis_name="s")

    @functools.partial(
        pl.kernel,
        out_type=[
            jax.ShapeDtypeStruct((NC, W, d), jnp.float32),
            jax.ShapeDtypeStruct((NC, NS, W), jnp.float32),
            jax.ShapeDtypeStruct((NC, NS, W), jnp.float32),
        ],
        mesh=mesh,
        scratch_types=[
            pltpu.VMEM((C,), jnp.int32),
            pltpu.VMEM((C,), jnp.int32),
            pltpu.VMEM((C, d), jnp.float32),
            pltpu.VMEM((C, d), jnp.float32),
            pltpu.VMEM((C, L), jnp.float32),
            pltpu.VMEM((d,), jnp.float32),
            pltpu.VMEM((W,), jnp.float32),
            pltpu.VMEM((W,), jnp.float32),
            pltpu.VMEM_SHARED((W, d), jnp.float32),
            pltpu.SemaphoreType.DMA,
            pltpu.SemaphoreType.DMA,
        ],
    )
    def k(xl_hbm, xr_hbm, att_hbm, src_hbm, dst_hbm, accr_hbm, den_hbm,
          cnt_hbm, srcv, dstv, xlv, xrv, exv, attv, denomv, countv, acc_sh,
          sem1, sem2):
        cid = lax.axis_index("c")
        sid = lax.axis_index("s")
        nbase = cid * half

        @pl.loop(0, ZD)
        def zbuf(i):
            for kk in range(dk):
                xlv[i, pl.ds(kk * L, L)] = jnp.zeros((L,), jnp.float32)

        @pl.loop(0, rpt // ZD)
        def zacc(j):
            off = sid * rpt + j * ZD
            pltpu.sync_copy(xlv.at[pl.ds(0, ZD)], acc_sh.at[pl.ds(off, ZD)])

        @pl.loop(0, W // L)
        def zdc(i):
            denomv[pl.ds(i * L, L)] = jnp.zeros((L,), jnp.float32)
            countv[pl.ds(i * L, L)] = jnp.zeros((L,), jnp.float32)

        plsc.subcore_barrier()

        pltpu.sync_copy(att_hbm, attv)
        att6 = [attv[pl.ds(kk * L, L)] * 0.6 for kk in range(dk)]
        att4 = [attv[pl.ds(kk * L, L)] * 0.4 for kk in range(dk)]
        ones = jnp.full((L,), 1.0, jnp.float32)
        zeros = jnp.zeros((L,), jnp.float32)
        lane = lax.iota(jnp.int32, L)
        trash = jnp.full((L,), TRASH, jnp.int32)

        @pl.loop(0, nchunk)
        def chunk_body(ci):
            base = sid * ew + ci * C
            pltpu.sync_copy(src_hbm.at[pl.ds(base, C)], srcv)
            pltpu.sync_copy(dst_hbm.at[pl.ds(base, C)], dstv)
            cp1 = pltpu.async_copy(xl_hbm.at[srcv], xlv, sem1)
            cp2 = pltpu.async_copy(xr_hbm.at[dstv], xrv, sem2)
            cp1.wait()
            cp2.wait()

            for g in range(ng):
                d16 = dstv[pl.ds(g * L, L)]
                loc = d16 - nbase
                own = (d16 >= nbase) & (loc < half)
                dstv[pl.ds(g * L, L)] = jnp.where(own, loc, trash)

            @plsc.parallel_loop(0, C)
            def edge_body(ei):
                parts = None
                xls = []
                for kk in range(dk):
                    a = xlv[ei, pl.ds(kk * L, L)]
                    b = xrv[ei, pl.ds(kk * L, L)]
                    v = a + b
                    t = att6[kk] * v + att4[kk] * jnp.abs(v)
                    parts = t if parts is None else parts + t
                    xls.append(a)
                s = parts
                for sh in (8, 4, 2, 1):
                    perm = jnp.bitwise_xor(lane, sh)
                    s = s + _lane_take(s, perm)
                ex = jnp.exp(s)
                exv[ei, :] = ex
                for kk in range(dk):
                    xlv[ei, pl.ds(kk * L, L)] = xls[kk] * ex

            for g in range(ng):
                d16 = dstv[pl.ds(g * L, L)]
                for j in range(L):
                    exj = exv[g * L + j, :]
                    loc = d16[j]
                    bs = lax.div(loc, L) * L
                    msk = lane == (loc - bs)
                    denomv[pl.ds(bs, L)] = (denomv[pl.ds(bs, L)]
                                            + jnp.where(msk, exj, zeros))
                    countv[pl.ds(bs, L)] = (countv[pl.ds(bs, L)]
                                            + jnp.where(msk, ones, zeros))

            pltpu.sync_copy(xlv, acc_sh.at[dstv], add=True)

        plsc.subcore_barrier()

        @pl.loop(0, rpt // ZD)
        def dump(j):
            off = sid * rpt + j * ZD
            pltpu.sync_copy(acc_sh.at[pl.ds(off, ZD)],
                            accr_hbm.at[cid, pl.ds(off, ZD)])

        pltpu.sync_copy(denomv, den_hbm.at[cid, sid])
        pltpu.sync_copy(countv, cnt_hbm.at[cid, sid])

    return k(xl, xr, att, src, dst)


def _fin_body(accr_ref, den_ref, cnt_ref, bias_ref, out_ref):
    r = accr_ref[0]
    denom = jnp.sum(den_ref[0], axis=1)[:, None] + 1e-16
    count = jnp.maximum(jnp.sum(cnt_ref[0], axis=1), 1.0)[:, None]
    out_ref[...] = r / (denom * count) + bias_ref[...]


def _finalize(accr, den_t, cnt_t, bias2d):
    n = 10000
    d = accr.shape[-1]
    bn = 200
    nb = (n // NC) // bn
    return pl.pallas_call(
        _fin_body,
        grid=(n // bn,),
        in_specs=[
            pl.BlockSpec((1, bn, d),
                         lambda i: (lax.div(i, nb), lax.rem(i, nb), 0)),
            pl.BlockSpec((1, bn, NS),
                         lambda i: (lax.div(i, nb), lax.rem(i, nb), 0)),
            pl.BlockSpec((1, bn, NS),
                         lambda i: (lax.div(i, nb), lax.rem(i, nb), 0)),
            pl.BlockSpec((1, d), lambda i: (0, 0)),
        ],
        out_specs=pl.BlockSpec((bn, d), lambda i: (i, 0)),
        out_shape=jax.ShapeDtypeStruct((n, d), jnp.float32),
    )(accr, den_t, cnt_t, bias2d)


def kernel(x, edge_index, W_l, W_r, att, bias):
    src = edge_index[0].astype(jnp.int32)
    dst = edge_index[1].astype(jnp.int32)
    xl, xr = _dual_matmul(x, W_l, W_r)
    accr, den, cnt = _sc_edge(xl, xr, att, src, dst)
    den_t = jnp.transpose(den, (0, 2, 1))
    cnt_t = jnp.transpose(cnt, (0, 2, 1))
    return _finalize(accr, den_t, cnt_t, bias.reshape(1, -1))
